# baseline (speedup 1.0000x reference)
import jax
import jax.numpy as jnp
from jax.experimental import pallas as pl
from jax.experimental.pallas import tpu as pltpu

_FORWARD_WEIGHT = 1.0
_BACKWARD_WEIGHT = 1.0
_MB = 1024


def _chamfer_kernel(pred_ref, tgt_t_ref, loss_ref, fwd_scratch):
    b = pl.program_id(0)
    m = pl.program_id(1)
    num_m = pl.num_programs(1)

    px = pred_ref[0, :, 0:1]
    py = pred_ref[0, :, 1:2]
    pz = pred_ref[0, :, 2:3]
    tx = tgt_t_ref[0, 0:1, :]
    ty = tgt_t_ref[0, 1:2, :]
    tz = tgt_t_ref[0, 2:3, :]

    def bf(v):
        return v.astype(jnp.bfloat16).astype(jnp.float32)

    pn = px * px + py * py + pz * pz
    tn = tx * tx + ty * ty + tz * tz
    pt = bf(px) * bf(tx) + bf(py) * bf(ty) + bf(pz) * bf(tz)
    d2 = (pn + tn) - 2.0 * pt

    col_min = jnp.min(d2, axis=0, keepdims=True)
    bwd_sum = jnp.sum(jnp.sqrt(jnp.maximum(col_min, 1e-12)), keepdims=True)

    row_min = jnp.min(d2, axis=1, keepdims=True)

    @pl.when(jnp.logical_and(b == 0, m == 0))
    def _():
        loss_ref[:, :] = jnp.zeros((1, 1), jnp.float32)

    @pl.when(m == 0)
    def _():
        fwd_scratch[:, :] = row_min

    @pl.when(m > 0)
    def _():
        fwd_scratch[:, :] = jnp.minimum(fwd_scratch[:, :], row_min)

    loss_ref[:, :] += _BACKWARD_WEIGHT * bwd_sum

    @pl.when(m == num_m - 1)
    def _():
        fwd = jnp.sqrt(jnp.maximum(fwd_scratch[:, :], 1e-12))
        loss_ref[:, :] += _FORWARD_WEIGHT * jnp.sum(fwd, keepdims=True)


def kernel(pred, target):
    if pred.ndim == 2:
        pred = pred[None, ...]
    if target.ndim == 2:
        target = target[None, ...]
    B, N, D = pred.shape
    _, M, _ = target.shape
    tgt_t = jnp.swapaxes(target, 1, 2)
    mb = min(_MB, M)
    grid = (B, M // mb)
    loss = pl.pallas_call(
        _chamfer_kernel,
        grid=grid,
        in_specs=[
            pl.BlockSpec((1, N, D), lambda b, m: (b, 0, 0)),
            pl.BlockSpec((1, D, mb), lambda b, m: (b, 0, m)),
        ],
        out_specs=pl.BlockSpec((1, 1), lambda b, m: (0, 0)),
        out_shape=jax.ShapeDtypeStruct((1, 1), jnp.float32),
        scratch_shapes=[pltpu.VMEM((N, 1), jnp.float32)],
    )(pred, tgt_t)
    return loss[0, 0] / B


# speedup vs baseline: 1.7049x; 1.7049x over previous
import jax
import jax.numpy as jnp
from jax.experimental import pallas as pl
from jax.experimental.pallas import tpu as pltpu

_FORWARD_WEIGHT = 1.0
_BACKWARD_WEIGHT = 1.0
_MB = 1024


def _chamfer_kernel(pred_ref, tgt_t_ref, loss_ref, fwd_scratch):
    b = pl.program_id(0)
    m = pl.program_id(1)
    num_m = pl.num_programs(1)

    px = pred_ref[0, :, 0:1]
    py = pred_ref[0, :, 1:2]
    pz = pred_ref[0, :, 2:3]
    tx = tgt_t_ref[0, 0:1, :]
    ty = tgt_t_ref[0, 1:2, :]
    tz = tgt_t_ref[0, 2:3, :]

    pn = px * px + py * py + pz * pz
    tn = tx * tx + ty * ty + tz * tz
    pb = pred_ref[0, :, :].astype(jnp.bfloat16) * jnp.bfloat16(2.0)
    tb = tgt_t_ref[0, :, :].astype(jnp.bfloat16)
    pt2 = jax.lax.dot_general(pb, tb, (((1,), (0,)), ((), ())),
                              preferred_element_type=jnp.float32)
    d2 = (pn + tn) - pt2

    col_min = jnp.min(d2, axis=0, keepdims=True)
    bwd_sum = jnp.sum(jnp.sqrt(jnp.maximum(col_min, 1e-12)), keepdims=True)

    row_min = jnp.min(d2, axis=1, keepdims=True)

    @pl.when(jnp.logical_and(b == 0, m == 0))
    def _():
        loss_ref[:, :] = jnp.zeros((1, 1), jnp.float32)

    @pl.when(m == 0)
    def _():
        fwd_scratch[:, :] = row_min

    @pl.when(m > 0)
    def _():
        fwd_scratch[:, :] = jnp.minimum(fwd_scratch[:, :], row_min)

    loss_ref[:, :] += _BACKWARD_WEIGHT * bwd_sum

    @pl.when(m == num_m - 1)
    def _():
        fwd = jnp.sqrt(jnp.maximum(fwd_scratch[:, :], 1e-12))
        loss_ref[:, :] += _FORWARD_WEIGHT * jnp.sum(fwd, keepdims=True)


def kernel(pred, target):
    if pred.ndim == 2:
        pred = pred[None, ...]
    if target.ndim == 2:
        target = target[None, ...]
    B, N, D = pred.shape
    _, M, _ = target.shape
    tgt_t = jnp.swapaxes(target, 1, 2)
    mb = min(_MB, M)
    grid = (B, M // mb)
    loss = pl.pallas_call(
        _chamfer_kernel,
        grid=grid,
        in_specs=[
            pl.BlockSpec((1, N, D), lambda b, m: (b, 0, 0)),
            pl.BlockSpec((1, D, mb), lambda b, m: (b, 0, m)),
        ],
        out_specs=pl.BlockSpec((1, 1), lambda b, m: (0, 0)),
        out_shape=jax.ShapeDtypeStruct((1, 1), jnp.float32),
        scratch_shapes=[pltpu.VMEM((N, 1), jnp.float32)],
    )(pred, tgt_t)
    return loss[0, 0] / B


# speedup vs baseline: 2.2707x; 1.3319x over previous
import jax
import jax.numpy as jnp
from jax.experimental import pallas as pl
from jax.experimental.pallas import tpu as pltpu

_FORWARD_WEIGHT = 1.0
_BACKWARD_WEIGHT = 1.0
_MB = 1024


def _chamfer_kernel(pred_ref, tgt_t_ref, loss_ref, fwd_scratch, pn_scratch):
    b = pl.program_id(0)
    m = pl.program_id(1)
    num_m = pl.num_programs(1)

    @pl.when(m == 0)
    def _():
        px = pred_ref[0, :, 0:1]
        py = pred_ref[0, :, 1:2]
        pz = pred_ref[0, :, 2:3]
        pn_scratch[:, :] = px * px + py * py + pz * pz

    tx = tgt_t_ref[0, 0:1, :]
    ty = tgt_t_ref[0, 1:2, :]
    tz = tgt_t_ref[0, 2:3, :]
    tn = tx * tx + ty * ty + tz * tz

    pb = pred_ref[0, :, :].astype(jnp.bfloat16) * jnp.bfloat16(2.0)
    tb = tgt_t_ref[0, :, :].astype(jnp.bfloat16)
    pt2 = jax.lax.dot_general(pb, tb, (((1,), (0,)), ((), ())),
                              preferred_element_type=jnp.float32)

    pn = pn_scratch[:, :]

    col_min = jnp.min(pn - pt2, axis=0, keepdims=True) + tn
    bwd_sum = jnp.sum(jnp.sqrt(jnp.maximum(col_min, 1e-12)), keepdims=True)

    row_min = jnp.min(tn - pt2, axis=1, keepdims=True)

    @pl.when(jnp.logical_and(b == 0, m == 0))
    def _():
        loss_ref[:, :] = jnp.zeros((1, 1), jnp.float32)

    @pl.when(m == 0)
    def _():
        fwd_scratch[:, :] = row_min

    @pl.when(m > 0)
    def _():
        fwd_scratch[:, :] = jnp.minimum(fwd_scratch[:, :], row_min)

    loss_ref[:, :] += _BACKWARD_WEIGHT * bwd_sum

    @pl.when(m == num_m - 1)
    def _():
        fwd = jnp.sqrt(jnp.maximum(fwd_scratch[:, :] + pn, 1e-12))
        loss_ref[:, :] += _FORWARD_WEIGHT * jnp.sum(fwd, keepdims=True)


def kernel(pred, target):
    if pred.ndim == 2:
        pred = pred[None, ...]
    if target.ndim == 2:
        target = target[None, ...]
    B, N, D = pred.shape
    _, M, _ = target.shape
    tgt_t = jnp.swapaxes(target, 1, 2)
    mb = min(_MB, M)
    grid = (B, M // mb)
    loss = pl.pallas_call(
        _chamfer_kernel,
        grid=grid,
        in_specs=[
            pl.BlockSpec((1, N, D), lambda b, m: (b, 0, 0)),
            pl.BlockSpec((1, D, mb), lambda b, m: (b, 0, m)),
        ],
        out_specs=pl.BlockSpec((1, 1), lambda b, m: (0, 0)),
        out_shape=jax.ShapeDtypeStruct((1, 1), jnp.float32),
        scratch_shapes=[
            pltpu.VMEM((N, 1), jnp.float32),
            pltpu.VMEM((N, 1), jnp.float32),
        ],
    )(pred, tgt_t)
    return loss[0, 0] / B
